# Initial kernel scaffold; baseline (speedup 1.0000x reference)
import jax
import jax.numpy as jnp
from jax.experimental import pallas as pl


def kernel(x, edge_index, num_rounds, W, b, gamma, beta):
    raise NotImplementedError("write your pallas kernel here")


# speedup vs baseline: 10.5565x; 10.5565x over previous
import functools

import jax
import jax.numpy as jnp
from jax import lax
from jax.experimental import pallas as pl
from jax.experimental.pallas import tpu as pltpu
from jax.experimental.pallas import tpu_sc as plsc

_EPS = 1e-5
_NC = 2
_NS = 16
_C = 80


def _sc_mesh():
    return plsc.VectorSubcoreMesh(core_axis_name="c", subcore_axis_name="s")


def _make_deg_kernel(NP, D, E):
    EP = E // (_NC * _NS)
    NCH = EP // _C
    RP = NP // _NS

    @functools.partial(
        pl.kernel,
        mesh=_sc_mesh(),
        out_type=jax.ShapeDtypeStruct((_NC, NP, D), jnp.float32),
        scratch_types=[
            pltpu.VMEM((_C,), jnp.int32),
            pltpu.VMEM((_C, D), jnp.float32),
            pltpu.VMEM_SHARED((NP, D), jnp.float32),
        ],
    )
    def deg_kernel(dst_hbm, ones_hbm, z_hbm, out_hbm, idx_v, ones_v, deg_sp):
        c = lax.axis_index("c")
        s = lax.axis_index("s")
        w = c * _NS + s
        slab = pl.ds(s * RP, RP)
        pltpu.sync_copy(ones_hbm, ones_v)
        pltpu.sync_copy(z_hbm, deg_sp.at[slab])
        plsc.subcore_barrier()

        def chunk(j, carry):
            pltpu.sync_copy(dst_hbm.at[pl.ds(w * EP + j * _C, _C)], idx_v)
            pltpu.sync_copy(ones_v, deg_sp.at[idx_v], add=True)
            return carry

        lax.fori_loop(0, NCH, chunk, None)
        plsc.subcore_barrier()
        pltpu.sync_copy(deg_sp.at[slab], out_hbm.at[c, slab])

    return deg_kernel


def _make_edge_kernel(NP, D, E):
    EP = E // (_NC * _NS)
    NCH = EP // _C
    RP = NP // _NS

    @functools.partial(
        pl.kernel,
        mesh=_sc_mesh(),
        out_type=jax.ShapeDtypeStruct((_NC, NP, D), jnp.float32),
        scratch_types=[
            pltpu.VMEM((_C,), jnp.int32),
            pltpu.VMEM((_C,), jnp.int32),
            pltpu.VMEM((_C, D), jnp.float32),
            pltpu.VMEM_SHARED((NP, D), jnp.float32),
            pltpu.SemaphoreType.DMA,
        ],
    )
    def edge_kernel(s_hbm, src_hbm, dst_hbm, z_hbm, out_hbm,
                    si_v, di_v, rows_v, acc_sp, sem):
        c = lax.axis_index("c")
        s = lax.axis_index("s")
        w = c * _NS + s
        slab = pl.ds(s * RP, RP)

        @pl.when(c == 0)
        def _():
            pltpu.sync_copy(s_hbm.at[slab], acc_sp.at[slab])

        @pl.when(c != 0)
        def _():
            pltpu.sync_copy(z_hbm, acc_sp.at[slab])

        plsc.subcore_barrier()

        def chunk(j, carry):
            pltpu.sync_copy(src_hbm.at[pl.ds(w * EP + j * _C, _C)], si_v)
            pltpu.sync_copy(dst_hbm.at[pl.ds(w * EP + j * _C, _C)], di_v)
            pltpu.async_copy(s_hbm.at[si_v], rows_v, sem).wait()
            pltpu.sync_copy(rows_v, acc_sp.at[di_v], add=True)
            return carry

        lax.fori_loop(0, NCH, chunk, None)
        plsc.subcore_barrier()
        pltpu.sync_copy(acc_sp.at[slab], out_hbm.at[c, slab])

    return edge_kernel


def _make_tc1(NP, D, BN=1024):
    grid = NP // BN

    def tc1_body(deg_ref, h_ref, w_ref, s_ref):
        d = deg_ref[0] + deg_ref[1]
        dis = lax.rsqrt(d[:, 0:1] + 1.0)
        xw = jnp.dot(h_ref[...], w_ref[...], preferred_element_type=jnp.float32,
                     precision=lax.Precision.HIGHEST)
        s_ref[...] = xw * dis

    return pl.pallas_call(
        tc1_body,
        grid=(grid,),
        in_specs=[
            pl.BlockSpec((_NC, BN, D), lambda i: (0, i, 0)),
            pl.BlockSpec((BN, D), lambda i: (i, 0)),
            pl.BlockSpec((D, D), lambda i: (0, 0)),
        ],
        out_specs=pl.BlockSpec((BN, D), lambda i: (i, 0)),
        out_shape=jax.ShapeDtypeStruct((NP, D), jnp.float32),
    )


def _make_tc2(NP, D, BN=1024):
    grid = NP // BN

    def tc2_body(acc_ref, deg_ref, b_ref, g_ref, be_ref, h_ref):
        d = deg_ref[0] + deg_ref[1]
        dis = lax.rsqrt(d[:, 0:1] + 1.0)
        t = (acc_ref[0] + acc_ref[1]) * dis + b_ref[...]
        mu = jnp.mean(t, axis=1, keepdims=True)
        tcen = t - mu
        var = jnp.mean(tcen * tcen, axis=1, keepdims=True)
        h_ref[...] = tcen * lax.rsqrt(var + _EPS) * g_ref[...] + be_ref[...]

    return pl.pallas_call(
        tc2_body,
        grid=(grid,),
        in_specs=[
            pl.BlockSpec((_NC, BN, D), lambda i: (0, i, 0)),
            pl.BlockSpec((_NC, BN, D), lambda i: (0, i, 0)),
            pl.BlockSpec((1, D), lambda i: (0, 0)),
            pl.BlockSpec((1, D), lambda i: (0, 0)),
            pl.BlockSpec((1, D), lambda i: (0, 0)),
        ],
        out_specs=pl.BlockSpec((BN, D), lambda i: (i, 0)),
        out_shape=jax.ShapeDtypeStruct((NP, D), jnp.float32),
    )


def kernel(x, edge_index, num_rounds, W, b, gamma, beta):
    N, D = x.shape
    E = edge_index.shape[1]
    ALIGN = 1024
    NP = (N + ALIGN - 1) // ALIGN * ALIGN
    RP = NP // _NS

    src1 = edge_index[0].astype(jnp.int32)
    dst1 = edge_index[1].astype(jnp.int32)
    onesrows = jnp.ones((_C, D), jnp.float32)
    zrows = jnp.zeros((RP, D), jnp.float32)
    xp = jnp.zeros((NP, D), jnp.float32).at[:N].set(x)
    b2 = b.reshape(1, D)
    g2 = gamma.reshape(1, D)
    be2 = beta.reshape(1, D)

    deg_kernel = _make_deg_kernel(NP, D, E)
    edge_kernel = _make_edge_kernel(NP, D, E)
    tc1 = _make_tc1(NP, D)
    tc2 = _make_tc2(NP, D)

    deg = deg_kernel(dst1, onesrows, zrows)

    def round_body(_, h):
        sarr = tc1(deg, h, W)
        acc = edge_kernel(sarr, src1, dst1, zrows)
        return tc2(acc, deg, b2, g2, be2)

    return lax.fori_loop(0, num_rounds, round_body, xp)[:N]


# speedup vs baseline: 17.4323x; 1.6513x over previous
import functools

import jax
import jax.numpy as jnp
from jax import lax
from jax.experimental import pallas as pl
from jax.experimental.pallas import tpu as pltpu
from jax.experimental.pallas import tpu_sc as plsc

_EPS = 1e-5
_NC = 2
_NS = 16
_C = 40


def _sc_mesh():
    return plsc.VectorSubcoreMesh(core_axis_name="c", subcore_axis_name="s")


def _make_deg_kernel(NP, D, E):
    EP = E // (_NC * _NS)
    NCH = EP // _C
    RP = NP // _NS

    @functools.partial(
        pl.kernel,
        mesh=_sc_mesh(),
        out_type=jax.ShapeDtypeStruct((_NC, NP, D), jnp.float32),
        scratch_types=[
            pltpu.VMEM((_C,), jnp.int32),
            pltpu.VMEM((_C, D), jnp.float32),
            pltpu.VMEM_SHARED((NP, D), jnp.float32),
        ],
    )
    def deg_kernel(dst_hbm, ones_hbm, z_hbm, out_hbm, idx_v, ones_v, deg_sp):
        c = lax.axis_index("c")
        s = lax.axis_index("s")
        w = c * _NS + s
        slab = pl.ds(s * RP, RP)
        pltpu.sync_copy(ones_hbm, ones_v)
        pltpu.sync_copy(z_hbm, deg_sp.at[slab])
        plsc.subcore_barrier()

        def chunk(j, carry):
            pltpu.sync_copy(dst_hbm.at[pl.ds(w * EP + j * _C, _C)], idx_v)
            pltpu.sync_copy(ones_v, deg_sp.at[idx_v], add=True)
            return carry

        lax.fori_loop(0, NCH, chunk, None)
        plsc.subcore_barrier()
        pltpu.sync_copy(deg_sp.at[slab], out_hbm.at[c, slab])

    return deg_kernel


def _make_edge_kernel(NP, D, E):
    NB = 5
    EP = E // (_NC * _NS)
    NCH = EP // _C
    RP = NP // _NS
    assert NCH % NB == 0

    @functools.partial(
        pl.kernel,
        mesh=_sc_mesh(),
        out_type=jax.ShapeDtypeStruct((_NC, NP, D), jnp.float32),
        scratch_types=(
            [pltpu.VMEM((_C,), jnp.int32) for _ in range(NB)]
            + [pltpu.VMEM((_C,), jnp.int32) for _ in range(NB)]
            + [pltpu.VMEM((_C, D), jnp.float32) for _ in range(NB)]
            + [
                pltpu.VMEM_SHARED((NP, D), jnp.float32),
                pltpu.SemaphoreType.DMA((NB,)),
                pltpu.SemaphoreType.DMA((NB,)),
                pltpu.SemaphoreType.DMA((NB,)),
            ]
        ),
    )
    def edge_kernel(s_hbm, src_hbm, dst_hbm, z_hbm, out_hbm, *scratch):
        si = scratch[:NB]
        di = scratch[NB:2 * NB]
        rows = scratch[2 * NB:3 * NB]
        acc_sp, isem, gsem, ssem = scratch[3 * NB:]
        c = lax.axis_index("c")
        s = lax.axis_index("s")
        w = c * _NS + s
        slab = pl.ds(s * RP, RP)
        base = w * EP

        def idx_start(m, b):
            pltpu.async_copy(src_hbm.at[pl.ds(base + m * _C, _C)], si[b],
                             isem.at[b])
            pltpu.async_copy(dst_hbm.at[pl.ds(base + m * _C, _C)], di[b],
                             isem.at[b])

        def idx_wait(b):
            pltpu.make_async_copy(src_hbm.at[pl.ds(0, _C)], si[b],
                                  isem.at[b]).wait()
            pltpu.make_async_copy(dst_hbm.at[pl.ds(0, _C)], di[b],
                                  isem.at[b]).wait()

        def gather_start(b):
            pltpu.async_copy(s_hbm.at[si[b]], rows[b], gsem.at[b])

        def gather_wait(b):
            pltpu.make_async_copy(s_hbm.at[si[b]], rows[b], gsem.at[b]).wait()

        def scat_start(b):
            pltpu.async_copy(rows[b], acc_sp.at[di[b]], ssem.at[b], add=True)

        def scat_wait(b):
            pltpu.make_async_copy(rows[b], acc_sp.at[di[b]], ssem.at[b]).wait()

        @pl.when(c == 0)
        def _():
            pltpu.sync_copy(s_hbm.at[slab], acc_sp.at[slab])

        @pl.when(c != 0)
        def _():
            pltpu.sync_copy(z_hbm, acc_sp.at[slab])

        plsc.subcore_barrier()

        for m in range(3):
            idx_start(m, m)
        for m in range(2):
            idx_wait(m)
            gather_start(m)

        def group(g, carry):
            for b in range(NB):
                j = g * NB + b
                gather_wait(b)
                scat_start(b)
                b3 = (b + 3) % NB

                @pl.when(j + 3 < NCH)
                def _(j=j, b3=b3):
                    @pl.when(j >= 2)
                    def _():
                        scat_wait(b3)

                    idx_start(j + 3, b3)

                b2 = (b + 2) % NB

                @pl.when(j + 2 < NCH)
                def _(b2=b2):
                    idx_wait(b2)
                    gather_start(b2)

            return carry

        lax.fori_loop(0, NCH // NB, group, None)
        for b in range(NB):
            scat_wait(b)
        plsc.subcore_barrier()
        pltpu.sync_copy(acc_sp.at[slab], out_hbm.at[c, slab])

    return edge_kernel


def _make_tc1(NP, D, BN=1024):
    grid = NP // BN

    def tc1_body(deg_ref, h_ref, w_ref, s_ref):
        d = deg_ref[0] + deg_ref[1]
        dis = lax.rsqrt(d[:, 0:1] + 1.0)
        xw = jnp.dot(h_ref[...], w_ref[...], preferred_element_type=jnp.float32,
                     precision=lax.Precision.HIGHEST)
        s_ref[...] = xw * dis

    return pl.pallas_call(
        tc1_body,
        grid=(grid,),
        in_specs=[
            pl.BlockSpec((_NC, BN, D), lambda i: (0, i, 0)),
            pl.BlockSpec((BN, D), lambda i: (i, 0)),
            pl.BlockSpec((D, D), lambda i: (0, 0)),
        ],
        out_specs=pl.BlockSpec((BN, D), lambda i: (i, 0)),
        out_shape=jax.ShapeDtypeStruct((NP, D), jnp.float32),
    )


def _make_tc2(NP, D, BN=1024):
    grid = NP // BN

    def tc2_body(acc_ref, deg_ref, b_ref, g_ref, be_ref, h_ref):
        d = deg_ref[0] + deg_ref[1]
        dis = lax.rsqrt(d[:, 0:1] + 1.0)
        t = (acc_ref[0] + acc_ref[1]) * dis + b_ref[...]
        mu = jnp.mean(t, axis=1, keepdims=True)
        tcen = t - mu
        var = jnp.mean(tcen * tcen, axis=1, keepdims=True)
        h_ref[...] = tcen * lax.rsqrt(var + _EPS) * g_ref[...] + be_ref[...]

    return pl.pallas_call(
        tc2_body,
        grid=(grid,),
        in_specs=[
            pl.BlockSpec((_NC, BN, D), lambda i: (0, i, 0)),
            pl.BlockSpec((_NC, BN, D), lambda i: (0, i, 0)),
            pl.BlockSpec((1, D), lambda i: (0, 0)),
            pl.BlockSpec((1, D), lambda i: (0, 0)),
            pl.BlockSpec((1, D), lambda i: (0, 0)),
        ],
        out_specs=pl.BlockSpec((BN, D), lambda i: (i, 0)),
        out_shape=jax.ShapeDtypeStruct((NP, D), jnp.float32),
    )


def kernel(x, edge_index, num_rounds, W, b, gamma, beta):
    N, D = x.shape
    E = edge_index.shape[1]
    ALIGN = 1024
    NP = (N + ALIGN - 1) // ALIGN * ALIGN
    RP = NP // _NS

    src1 = edge_index[0].astype(jnp.int32)
    dst1 = edge_index[1].astype(jnp.int32)
    onesrows = jnp.ones((_C, D), jnp.float32)
    zrows = jnp.zeros((RP, D), jnp.float32)
    xp = jnp.zeros((NP, D), jnp.float32).at[:N].set(x)
    b2 = b.reshape(1, D)
    g2 = gamma.reshape(1, D)
    be2 = beta.reshape(1, D)

    deg_kernel = _make_deg_kernel(NP, D, E)
    edge_kernel = _make_edge_kernel(NP, D, E)
    tc1 = _make_tc1(NP, D)
    tc2 = _make_tc2(NP, D)

    deg = deg_kernel(dst1, onesrows, zrows)

    def round_body(_, h):
        sarr = tc1(deg, h, W)
        acc = edge_kernel(sarr, src1, dst1, zrows)
        return tc2(acc, deg, b2, g2, be2)

    return lax.fori_loop(0, num_rounds, round_body, xp)[:N]


# speedup vs baseline: 23.8592x; 1.3687x over previous
import functools

import jax
import jax.numpy as jnp
from jax import lax
from jax.experimental import pallas as pl
from jax.experimental.pallas import tpu as pltpu
from jax.experimental.pallas import tpu_sc as plsc

_EPS = 1e-5
_NC = 2
_NS = 16
_C = 40


def _sc_mesh():
    return plsc.VectorSubcoreMesh(core_axis_name="c", subcore_axis_name="s")


def _make_deg_kernel(NP, D, E):
    CD = 80
    NB = 5
    EP = E // (_NC * _NS)
    NCH = EP // CD
    RP = NP // _NS
    assert NCH % NB == 0

    @functools.partial(
        pl.kernel,
        mesh=_sc_mesh(),
        out_type=jax.ShapeDtypeStruct((_NC, NP, D), jnp.float32),
        scratch_types=(
            [pltpu.VMEM((CD,), jnp.int32) for _ in range(NB)]
            + [
                pltpu.VMEM((CD, D), jnp.float32),
                pltpu.VMEM_SHARED((NP, D), jnp.float32),
                pltpu.SemaphoreType.DMA((NB,)),
                pltpu.SemaphoreType.DMA((NB,)),
            ]
        ),
    )
    def deg_kernel(dst_hbm, ones_hbm, z_hbm, out_hbm, *scratch):
        di = scratch[:NB]
        ones_v, deg_sp, isem, ssem = scratch[NB:]
        c = lax.axis_index("c")
        s = lax.axis_index("s")
        w = c * _NS + s
        slab = pl.ds(s * RP, RP)
        base = w * EP

        def idx_start(m, b):
            pltpu.async_copy(dst_hbm.at[pl.ds(base + m * CD, CD)], di[b],
                             isem.at[b])

        def idx_wait(b):
            pltpu.make_async_copy(dst_hbm.at[pl.ds(0, CD)], di[b],
                                  isem.at[b]).wait()

        def scat_start(b):
            pltpu.async_copy(ones_v, deg_sp.at[di[b]], ssem.at[b], add=True)

        def scat_wait(b):
            pltpu.make_async_copy(ones_v, deg_sp.at[di[b]], ssem.at[b]).wait()

        pltpu.sync_copy(ones_hbm, ones_v)
        pltpu.sync_copy(z_hbm, deg_sp.at[slab])
        plsc.subcore_barrier()

        for m in range(3):
            idx_start(m, m)

        def step(g, carry):
            for b in range(NB):
                j = g * NB + b
                idx_wait(b)
                scat_start(b)
                b3 = (b + 3) % NB

                @pl.when(j + 3 < NCH)
                def _(j=j, b3=b3):
                    @pl.when(j >= 2)
                    def _():
                        scat_wait(b3)

                    idx_start(j + 3, b3)

            return carry

        lax.fori_loop(0, NCH // NB, step, None)
        for b in range(NB):
            scat_wait(b)
        plsc.subcore_barrier()
        pltpu.sync_copy(deg_sp.at[slab], out_hbm.at[c, slab])

    return deg_kernel


def _make_edge_kernel(NP, D, E):
    NB = 5
    NI = 10
    EP = E // (_NC * _NS)
    NCH = EP // _C
    RP = NP // _NS
    assert NCH % NI == 0

    @functools.partial(
        pl.kernel,
        mesh=_sc_mesh(),
        out_type=jax.ShapeDtypeStruct((_NC, NP, D), jnp.float32),
        scratch_types=(
            [pltpu.VMEM((_C,), jnp.int32) for _ in range(NI)]
            + [pltpu.VMEM((_C,), jnp.int32) for _ in range(NI)]
            + [pltpu.VMEM((_C, D), jnp.float32) for _ in range(NB)]
            + [
                pltpu.VMEM_SHARED((NP, D), jnp.float32),
                pltpu.SemaphoreType.DMA((NI,)),
                pltpu.SemaphoreType.DMA((NB,)),
                pltpu.SemaphoreType.DMA((NB,)),
            ]
        ),
    )
    def edge_kernel(s_hbm, src_hbm, dst_hbm, z_hbm, out_hbm, *scratch):
        si = scratch[:NI]
        di = scratch[NI:2 * NI]
        rows = scratch[2 * NI:2 * NI + NB]
        acc_sp, isem, gsem, ssem = scratch[2 * NI + NB:]
        c = lax.axis_index("c")
        s = lax.axis_index("s")
        w = c * _NS + s
        slab = pl.ds(s * RP, RP)
        base = w * EP

        def idx_start(m, b):
            pltpu.async_copy(src_hbm.at[pl.ds(base + m * _C, _C)], si[b],
                             isem.at[b])
            pltpu.async_copy(dst_hbm.at[pl.ds(base + m * _C, _C)], di[b],
                             isem.at[b])

        def idx_wait(b):
            pltpu.make_async_copy(src_hbm.at[pl.ds(0, _C)], si[b],
                                  isem.at[b]).wait()
            pltpu.make_async_copy(dst_hbm.at[pl.ds(0, _C)], di[b],
                                  isem.at[b]).wait()

        def gather_start(bi, br):
            pltpu.async_copy(s_hbm.at[si[bi]], rows[br], gsem.at[br])

        def gather_wait(bi, br):
            pltpu.make_async_copy(s_hbm.at[si[bi]], rows[br],
                                  gsem.at[br]).wait()

        def scat_start(bi, br):
            pltpu.async_copy(rows[br], acc_sp.at[di[bi]], ssem.at[br],
                             add=True)

        def scat_wait(bi, br):
            pltpu.make_async_copy(rows[br], acc_sp.at[di[bi]],
                                  ssem.at[br]).wait()

        @pl.when(c == 0)
        def _():
            pltpu.sync_copy(s_hbm.at[slab], acc_sp.at[slab])

        @pl.when(c != 0)
        def _():
            pltpu.sync_copy(z_hbm, acc_sp.at[slab])

        plsc.subcore_barrier()

        for m in range(8):
            idx_start(m, m)
        for m in range(3):
            idx_wait(m)
            gather_start(m, m)

        def group(g, carry):
            for b in range(NI):
                j = g * NI + b
                b5 = b % NB
                gather_wait(b, b5)
                scat_start(b, b5)
                bi3 = (b + 3) % NI
                br3 = (b + 3) % NB

                @pl.when(j + 3 < NCH)
                def _(j=j, bi3=bi3, br3=br3):
                    @pl.when(j >= 2)
                    def _():
                        scat_wait((bi3 + NB) % NI, br3)

                    idx_wait(bi3)
                    gather_start(bi3, br3)

                bi8 = (b + 8) % NI

                @pl.when(j + 8 < NCH)
                def _(j=j, bi8=bi8):
                    idx_start(j + 8, bi8)

            return carry

        lax.fori_loop(0, NCH // NI, group, None)
        for m in range(5):
            b = (NCH - 5 + m) % NI
            scat_wait(b, b % NB)
        plsc.subcore_barrier()
        pltpu.sync_copy(acc_sp.at[slab], out_hbm.at[c, slab])

    return edge_kernel


def _make_tc1(NP, D, BN=1024):
    grid = NP // BN

    def tc1_body(deg_ref, h_ref, w_ref, s_ref):
        d = deg_ref[0] + deg_ref[1]
        dis = lax.rsqrt(d[:, 0:1] + 1.0)
        xw = jnp.dot(h_ref[...], w_ref[...], preferred_element_type=jnp.float32,
                     precision=lax.Precision.HIGHEST)
        s_ref[...] = xw * dis

    return pl.pallas_call(
        tc1_body,
        grid=(grid,),
        in_specs=[
            pl.BlockSpec((_NC, BN, D), lambda i: (0, i, 0)),
            pl.BlockSpec((BN, D), lambda i: (i, 0)),
            pl.BlockSpec((D, D), lambda i: (0, 0)),
        ],
        out_specs=pl.BlockSpec((BN, D), lambda i: (i, 0)),
        out_shape=jax.ShapeDtypeStruct((NP, D), jnp.float32),
    )


def _make_tc2(NP, D, BN=1024):
    grid = NP // BN

    def tc2_body(acc_ref, deg_ref, b_ref, g_ref, be_ref, h_ref):
        d = deg_ref[0] + deg_ref[1]
        dis = lax.rsqrt(d[:, 0:1] + 1.0)
        t = (acc_ref[0] + acc_ref[1]) * dis + b_ref[...]
        mu = jnp.mean(t, axis=1, keepdims=True)
        tcen = t - mu
        var = jnp.mean(tcen * tcen, axis=1, keepdims=True)
        h_ref[...] = tcen * lax.rsqrt(var + _EPS) * g_ref[...] + be_ref[...]

    return pl.pallas_call(
        tc2_body,
        grid=(grid,),
        in_specs=[
            pl.BlockSpec((_NC, BN, D), lambda i: (0, i, 0)),
            pl.BlockSpec((_NC, BN, D), lambda i: (0, i, 0)),
            pl.BlockSpec((1, D), lambda i: (0, 0)),
            pl.BlockSpec((1, D), lambda i: (0, 0)),
            pl.BlockSpec((1, D), lambda i: (0, 0)),
        ],
        out_specs=pl.BlockSpec((BN, D), lambda i: (i, 0)),
        out_shape=jax.ShapeDtypeStruct((NP, D), jnp.float32),
    )


def kernel(x, edge_index, num_rounds, W, b, gamma, beta):
    N, D = x.shape
    E = edge_index.shape[1]
    ALIGN = 1024
    NP = (N + ALIGN - 1) // ALIGN * ALIGN
    RP = NP // _NS

    src1 = edge_index[0].astype(jnp.int32)
    dst1 = edge_index[1].astype(jnp.int32)
    onesrows = jnp.ones((80, D), jnp.float32)
    zrows = jnp.zeros((RP, D), jnp.float32)
    xp = jnp.zeros((NP, D), jnp.float32).at[:N].set(x)
    b2 = b.reshape(1, D)
    g2 = gamma.reshape(1, D)
    be2 = beta.reshape(1, D)

    deg_kernel = _make_deg_kernel(NP, D, E)
    edge_kernel = _make_edge_kernel(NP, D, E)
    tc1 = _make_tc1(NP, D)
    tc2 = _make_tc2(NP, D)

    deg = deg_kernel(dst1, onesrows, zrows)

    def round_body(_, h):
        sarr = tc1(deg, h, W)
        acc = edge_kernel(sarr, src1, dst1, zrows)
        return tc2(acc, deg, b2, g2, be2)

    return lax.fori_loop(0, num_rounds, round_body, xp)[:N]


# speedup vs baseline: 24.4841x; 1.0262x over previous
import functools

import jax
import jax.numpy as jnp
from jax import lax
from jax.experimental import pallas as pl
from jax.experimental.pallas import tpu as pltpu
from jax.experimental.pallas import tpu_sc as plsc

_EPS = 1e-5
_NC = 2
_NS = 16
_C = 40


def _sc_mesh():
    return plsc.VectorSubcoreMesh(core_axis_name="c", subcore_axis_name="s")


def _make_deg_kernel(NP, D, E):
    CD = 80
    NB = 5
    EP = E // (_NC * _NS)
    NCH = EP // CD
    RP = NP // _NS
    assert NCH % NB == 0

    @functools.partial(
        pl.kernel,
        mesh=_sc_mesh(),
        out_type=jax.ShapeDtypeStruct((_NC, NP, D), jnp.float32),
        scratch_types=(
            [pltpu.VMEM((CD,), jnp.int32) for _ in range(NB)]
            + [
                pltpu.VMEM((CD, D), jnp.float32),
                pltpu.VMEM_SHARED((NP, D), jnp.float32),
                pltpu.SemaphoreType.DMA((NB,)),
                pltpu.SemaphoreType.DMA((NB,)),
            ]
        ),
    )
    def deg_kernel(dst_hbm, ones_hbm, z_hbm, out_hbm, *scratch):
        di = scratch[:NB]
        ones_v, deg_sp, isem, ssem = scratch[NB:]
        c = lax.axis_index("c")
        s = lax.axis_index("s")
        w = c * _NS + s
        slab = pl.ds(s * RP, RP)
        base = w * EP

        def idx_start(m, b):
            pltpu.async_copy(dst_hbm.at[pl.ds(base + m * CD, CD)], di[b],
                             isem.at[b])

        def idx_wait(b):
            pltpu.make_async_copy(dst_hbm.at[pl.ds(0, CD)], di[b],
                                  isem.at[b]).wait()

        def scat_start(b):
            pltpu.async_copy(ones_v, deg_sp.at[di[b]], ssem.at[b], add=True)

        def scat_wait(b):
            pltpu.make_async_copy(ones_v, deg_sp.at[di[b]], ssem.at[b]).wait()

        pltpu.sync_copy(ones_hbm, ones_v)
        pltpu.sync_copy(z_hbm, deg_sp.at[slab])
        plsc.subcore_barrier()

        for m in range(3):
            idx_start(m, m)

        def step(g, carry):
            for b in range(NB):
                j = g * NB + b
                idx_wait(b)
                scat_start(b)
                b3 = (b + 3) % NB

                @pl.when(j + 3 < NCH)
                def _(j=j, b3=b3):
                    @pl.when(j >= 2)
                    def _():
                        scat_wait(b3)

                    idx_start(j + 3, b3)

            return carry

        lax.fori_loop(0, NCH // NB, step, None)
        for b in range(NB):
            scat_wait(b)
        plsc.subcore_barrier()
        pltpu.sync_copy(deg_sp.at[slab], out_hbm.at[c, slab])

    return deg_kernel


def _make_edge_kernel(NP, D, E):
    NB = 5
    NI = 10
    EP = E // (_NC * _NS)
    NCH = EP // _C
    RP = NP // _NS
    assert NCH % NI == 0

    @functools.partial(
        pl.kernel,
        mesh=_sc_mesh(),
        out_type=jax.ShapeDtypeStruct((_NC, NP, D), jnp.float32),
        scratch_types=(
            [pltpu.VMEM((_C,), jnp.int32) for _ in range(NI)]
            + [pltpu.VMEM((_C,), jnp.int32) for _ in range(NI)]
            + [pltpu.VMEM((_C, D), jnp.float32) for _ in range(NB)]
            + [
                pltpu.VMEM_SHARED((NP, D), jnp.float32),
                pltpu.SemaphoreType.DMA((NI,)),
                pltpu.SemaphoreType.DMA((NB,)),
                pltpu.SemaphoreType.DMA((NB,)),
            ]
        ),
    )
    def edge_kernel(s_hbm, src_hbm, dst_hbm, z_hbm, out_hbm, *scratch):
        si = scratch[:NI]
        di = scratch[NI:2 * NI]
        rows = scratch[2 * NI:2 * NI + NB]
        acc_sp, isem, gsem, ssem = scratch[2 * NI + NB:]
        c = lax.axis_index("c")
        s = lax.axis_index("s")
        w = c * _NS + s
        slab = pl.ds(s * RP, RP)
        base = w * EP

        def idx_start(m, b):
            pltpu.async_copy(src_hbm.at[pl.ds(base + m * _C, _C)], si[b],
                             isem.at[b])
            pltpu.async_copy(dst_hbm.at[pl.ds(base + m * _C, _C)], di[b],
                             isem.at[b])

        def idx_wait(b):
            pltpu.make_async_copy(src_hbm.at[pl.ds(0, _C)], si[b],
                                  isem.at[b]).wait()
            pltpu.make_async_copy(dst_hbm.at[pl.ds(0, _C)], di[b],
                                  isem.at[b]).wait()

        def gather_start(bi, br):
            pltpu.async_copy(s_hbm.at[si[bi]], rows[br], gsem.at[br])

        def gather_wait(bi, br):
            pltpu.make_async_copy(s_hbm.at[si[bi]], rows[br],
                                  gsem.at[br]).wait()

        def scat_start(bi, br):
            pltpu.async_copy(rows[br], acc_sp.at[di[bi]], ssem.at[br],
                             add=True)

        def scat_wait(bi, br):
            pltpu.make_async_copy(rows[br], acc_sp.at[di[bi]],
                                  ssem.at[br]).wait()

        @pl.when(c == 0)
        def _():
            pltpu.sync_copy(s_hbm.at[slab], acc_sp.at[slab])

        @pl.when(c != 0)
        def _():
            pltpu.sync_copy(z_hbm, acc_sp.at[slab])

        plsc.subcore_barrier()

        for m in range(8):
            idx_start(m, m)
        for m in range(3):
            idx_wait(m)
            gather_start(m, m)

        def group(g, carry):
            for b in range(NI):
                j = g * NI + b
                b5 = b % NB
                gather_wait(b, b5)
                scat_start(b, b5)
                bi3 = (b + 3) % NI
                br3 = (b + 3) % NB

                @pl.when(j + 3 < NCH)
                def _(j=j, bi3=bi3, br3=br3):
                    @pl.when(j >= 2)
                    def _():
                        scat_wait((bi3 + NB) % NI, br3)

                    idx_wait(bi3)
                    gather_start(bi3, br3)

                bi8 = (b + 8) % NI

                @pl.when(j + 8 < NCH)
                def _(j=j, bi8=bi8):
                    idx_start(j + 8, bi8)

            return carry

        lax.fori_loop(0, NCH // NI, group, None)
        for m in range(5):
            b = (NCH - 5 + m) % NI
            scat_wait(b, b % NB)
        plsc.subcore_barrier()
        pltpu.sync_copy(acc_sp.at[slab], out_hbm.at[c, slab])

    return edge_kernel


def _make_tc1(NP, D, BN=1024):
    grid = NP // BN

    def tc1_body(deg_ref, h_ref, w_ref, s_ref):
        d = deg_ref[0] + deg_ref[1]
        dis = lax.rsqrt(d[:, 0:1] + 1.0)
        xw = jnp.dot(h_ref[...], w_ref[...], preferred_element_type=jnp.float32,
                     precision=lax.Precision.HIGHEST)
        s_ref[...] = xw * dis

    return pl.pallas_call(
        tc1_body,
        grid=(grid,),
        in_specs=[
            pl.BlockSpec((_NC, BN, D), lambda i: (0, i, 0)),
            pl.BlockSpec((BN, D), lambda i: (i, 0)),
            pl.BlockSpec((D, D), lambda i: (0, 0)),
        ],
        out_specs=pl.BlockSpec((BN, D), lambda i: (i, 0)),
        out_shape=jax.ShapeDtypeStruct((NP, D), jnp.float32),
    )


def _make_tc_round(NP, D, BN=1024):
    grid = NP // BN

    def body(acc_ref, deg_ref, w_ref, b_ref, g_ref, be_ref, h_ref, s_ref):
        d = deg_ref[0][:, 0:1] + deg_ref[1][:, 0:1] + 1.0
        dis = lax.rsqrt(d)
        t = (acc_ref[0] + acc_ref[1]) * dis + b_ref[...]
        mu = jnp.mean(t, axis=1, keepdims=True)
        tcen = t - mu
        var = jnp.mean(tcen * tcen, axis=1, keepdims=True)
        h = tcen * lax.rsqrt(var + _EPS) * g_ref[...] + be_ref[...]
        h_ref[...] = h
        s_ref[...] = jnp.dot(h, w_ref[...], preferred_element_type=jnp.float32,
                             precision=lax.Precision.HIGHEST) * dis

    return pl.pallas_call(
        body,
        grid=(grid,),
        in_specs=[
            pl.BlockSpec((_NC, BN, D), lambda i: (0, i, 0)),
            pl.BlockSpec((_NC, BN, D), lambda i: (0, i, 0)),
            pl.BlockSpec((D, D), lambda i: (0, 0)),
            pl.BlockSpec((1, D), lambda i: (0, 0)),
            pl.BlockSpec((1, D), lambda i: (0, 0)),
            pl.BlockSpec((1, D), lambda i: (0, 0)),
        ],
        out_specs=[
            pl.BlockSpec((BN, D), lambda i: (i, 0)),
            pl.BlockSpec((BN, D), lambda i: (i, 0)),
        ],
        out_shape=[
            jax.ShapeDtypeStruct((NP, D), jnp.float32),
            jax.ShapeDtypeStruct((NP, D), jnp.float32),
        ],
    )


def kernel(x, edge_index, num_rounds, W, b, gamma, beta):
    N, D = x.shape
    E = edge_index.shape[1]
    ALIGN = 1024
    NP = (N + ALIGN - 1) // ALIGN * ALIGN
    RP = NP // _NS

    src1 = edge_index[0].astype(jnp.int32)
    dst1 = edge_index[1].astype(jnp.int32)
    onesrows = jnp.ones((80, D), jnp.float32)
    zrows = jnp.zeros((RP, D), jnp.float32)
    xp = jnp.zeros((NP, D), jnp.float32).at[:N].set(x)
    b2 = b.reshape(1, D)
    g2 = gamma.reshape(1, D)
    be2 = beta.reshape(1, D)

    deg_kernel = _make_deg_kernel(NP, D, E)
    edge_kernel = _make_edge_kernel(NP, D, E)
    tc1 = _make_tc1(NP, D)
    tc_round = _make_tc_round(NP, D)

    deg = deg_kernel(dst1, onesrows, zrows)
    s0 = tc1(deg, xp, W)

    def round_body(_, carry):
        h, sarr = carry
        acc = edge_kernel(sarr, src1, dst1, zrows)
        h, sarr = tc_round(acc, deg, W, b2, g2, be2)
        return (h, sarr)

    h, _ = lax.fori_loop(0, num_rounds, round_body, (xp, s0))
    return h[:N]


# speedup vs baseline: 24.6774x; 1.0079x over previous
import functools

import jax
import jax.numpy as jnp
from jax import lax
from jax.experimental import pallas as pl
from jax.experimental.pallas import tpu as pltpu
from jax.experimental.pallas import tpu_sc as plsc

_EPS = 1e-5
_NC = 2
_NS = 16
_C = 40


def _sc_mesh():
    return plsc.VectorSubcoreMesh(core_axis_name="c", subcore_axis_name="s")


def _make_deg_kernel(NP, D, E):
    CD = 80
    NB = 5
    EP = E // (_NC * _NS)
    NCH = EP // CD
    RP = NP // _NS
    assert NCH % NB == 0

    @functools.partial(
        pl.kernel,
        mesh=_sc_mesh(),
        out_type=jax.ShapeDtypeStruct((_NC, NP, D), jnp.float32),
        scratch_types=(
            [pltpu.VMEM((CD,), jnp.int32) for _ in range(NB)]
            + [
                pltpu.VMEM((CD, D), jnp.float32),
                pltpu.VMEM_SHARED((NP, D), jnp.float32),
                pltpu.SemaphoreType.DMA((NB,)),
                pltpu.SemaphoreType.DMA((NB,)),
            ]
        ),
    )
    def deg_kernel(dst_hbm, ones_hbm, z_hbm, out_hbm, *scratch):
        di = scratch[:NB]
        ones_v, deg_sp, isem, ssem = scratch[NB:]
        c = lax.axis_index("c")
        s = lax.axis_index("s")
        w = c * _NS + s
        slab = pl.ds(s * RP, RP)
        base = w * EP

        def idx_start(m, b):
            pltpu.async_copy(dst_hbm.at[pl.ds(base + m * CD, CD)], di[b],
                             isem.at[b])

        def idx_wait(b):
            pltpu.make_async_copy(dst_hbm.at[pl.ds(0, CD)], di[b],
                                  isem.at[b]).wait()

        def scat_start(b):
            pltpu.async_copy(ones_v, deg_sp.at[di[b]], ssem.at[b], add=True)

        def scat_wait(b):
            pltpu.make_async_copy(ones_v, deg_sp.at[di[b]], ssem.at[b]).wait()

        pltpu.sync_copy(ones_hbm, ones_v)
        pltpu.sync_copy(z_hbm, deg_sp.at[slab])
        plsc.subcore_barrier()

        for m in range(3):
            idx_start(m, m)

        def step(g, carry):
            for b in range(NB):
                j = g * NB + b
                idx_wait(b)
                scat_start(b)
                b3 = (b + 3) % NB

                @pl.when(j + 3 < NCH)
                def _(j=j, b3=b3):
                    @pl.when(j >= 2)
                    def _():
                        scat_wait(b3)

                    idx_start(j + 3, b3)

            return carry

        lax.fori_loop(0, NCH // NB, step, None)
        for b in range(NB):
            scat_wait(b)
        plsc.subcore_barrier()
        pltpu.sync_copy(deg_sp.at[slab], out_hbm.at[c, slab])

    return deg_kernel


def _make_edge_kernel(NP, D, E):
    NB = 5
    NI = 10
    EP = E // (_NC * _NS)
    NCH = EP // _C
    RP = NP // _NS
    assert NCH % NI == 0

    @functools.partial(
        pl.kernel,
        mesh=_sc_mesh(),
        out_type=jax.ShapeDtypeStruct((_NC, NP, D), jnp.float32),
        scratch_types=(
            [pltpu.VMEM((_C,), jnp.int32) for _ in range(NI)]
            + [pltpu.VMEM((_C,), jnp.int32) for _ in range(NI)]
            + [pltpu.VMEM((_C, D), jnp.float32) for _ in range(NB)]
            + [
                pltpu.VMEM_SHARED((NP, D), jnp.float32),
                pltpu.SemaphoreType.DMA((NI,)),
                pltpu.SemaphoreType.DMA((NB,)),
                pltpu.SemaphoreType.DMA((NB,)),
            ]
        ),
    )
    def edge_kernel(s_hbm, src_hbm, dst_hbm, z_hbm, out_hbm, *scratch):
        si = scratch[:NI]
        di = scratch[NI:2 * NI]
        rows = scratch[2 * NI:2 * NI + NB]
        acc_sp, isem, gsem, ssem = scratch[2 * NI + NB:]
        c = lax.axis_index("c")
        s = lax.axis_index("s")
        w = c * _NS + s
        slab = pl.ds(s * RP, RP)
        base = w * EP

        def idx_start(m, b):
            pltpu.async_copy(src_hbm.at[pl.ds(base + m * _C, _C)], si[b],
                             isem.at[b])
            pltpu.async_copy(dst_hbm.at[pl.ds(base + m * _C, _C)], di[b],
                             isem.at[b])

        def idx_wait(b):
            pltpu.make_async_copy(src_hbm.at[pl.ds(0, _C)], si[b],
                                  isem.at[b]).wait()
            pltpu.make_async_copy(dst_hbm.at[pl.ds(0, _C)], di[b],
                                  isem.at[b]).wait()

        def gather_start(bi, br):
            pltpu.async_copy(s_hbm.at[si[bi]], rows[br], gsem.at[br])

        def gather_wait(bi, br):
            pltpu.make_async_copy(s_hbm.at[si[bi]], rows[br],
                                  gsem.at[br]).wait()

        def scat_start(bi, br):
            pltpu.async_copy(rows[br], acc_sp.at[di[bi]], ssem.at[br],
                             add=True)

        def scat_wait(bi, br):
            pltpu.make_async_copy(rows[br], acc_sp.at[di[bi]],
                                  ssem.at[br]).wait()

        @pl.when(c == 0)
        def _():
            pltpu.sync_copy(s_hbm.at[slab], acc_sp.at[slab])

        @pl.when(c != 0)
        def _():
            pltpu.sync_copy(z_hbm, acc_sp.at[slab])

        plsc.subcore_barrier()

        for m in range(8):
            idx_start(m, m)
        for m in range(3):
            idx_wait(m)
            gather_start(m, m)

        def group(g, carry):
            for b in range(NI):
                j = g * NI + b
                b5 = b % NB
                gather_wait(b, b5)
                scat_start(b, b5)
                bi3 = (b + 3) % NI
                br3 = (b + 3) % NB

                @pl.when(j + 3 < NCH)
                def _(j=j, bi3=bi3, br3=br3):
                    @pl.when(j >= 2)
                    def _():
                        scat_wait((bi3 + NB) % NI, br3)

                    idx_wait(bi3)
                    gather_start(bi3, br3)

                bi8 = (b + 8) % NI

                @pl.when(j + 8 < NCH)
                def _(j=j, bi8=bi8):
                    idx_start(j + 8, bi8)

            return carry

        lax.fori_loop(0, NCH // NI, group, None)
        for m in range(5):
            b = (NCH - 5 + m) % NI
            scat_wait(b, b % NB)
        plsc.subcore_barrier()
        pltpu.sync_copy(acc_sp.at[slab], out_hbm.at[c, slab])

    return edge_kernel


def _make_mm(NP, D, BN=1024):
    grid = NP // BN

    def body(h_ref, w_ref, o_ref):
        o_ref[...] = jnp.dot(h_ref[...], w_ref[...],
                             preferred_element_type=jnp.float32,
                             precision=lax.Precision.HIGHEST)

    return pl.pallas_call(
        body,
        grid=(grid,),
        in_specs=[
            pl.BlockSpec((BN, D), lambda i: (i, 0)),
            pl.BlockSpec((D, D), lambda i: (0, 0)),
        ],
        out_specs=pl.BlockSpec((BN, D), lambda i: (i, 0)),
        out_shape=jax.ShapeDtypeStruct((NP, D), jnp.float32),
    )


def _make_scale(NP, D, BN=1024):
    grid = NP // BN

    def body(deg_ref, xw_ref, s_ref):
        d = deg_ref[0][:, 0:1] + deg_ref[1][:, 0:1] + 1.0
        s_ref[...] = xw_ref[...] * lax.rsqrt(d)

    return pl.pallas_call(
        body,
        grid=(grid,),
        in_specs=[
            pl.BlockSpec((_NC, BN, D), lambda i: (0, i, 0)),
            pl.BlockSpec((BN, D), lambda i: (i, 0)),
        ],
        out_specs=pl.BlockSpec((BN, D), lambda i: (i, 0)),
        out_shape=jax.ShapeDtypeStruct((NP, D), jnp.float32),
    )


def _make_tc_round(NP, D, BN=1024):
    grid = NP // BN

    def body(acc_ref, deg_ref, w_ref, b_ref, g_ref, be_ref, h_ref, s_ref):
        d = deg_ref[0][:, 0:1] + deg_ref[1][:, 0:1] + 1.0
        dis = lax.rsqrt(d)
        t = (acc_ref[0] + acc_ref[1]) * dis + b_ref[...]
        mu = jnp.mean(t, axis=1, keepdims=True)
        tcen = t - mu
        var = jnp.mean(tcen * tcen, axis=1, keepdims=True)
        h = tcen * lax.rsqrt(var + _EPS) * g_ref[...] + be_ref[...]
        h_ref[...] = h
        s_ref[...] = jnp.dot(h, w_ref[...], preferred_element_type=jnp.float32,
                             precision=lax.Precision.HIGHEST) * dis

    return pl.pallas_call(
        body,
        grid=(grid,),
        in_specs=[
            pl.BlockSpec((_NC, BN, D), lambda i: (0, i, 0)),
            pl.BlockSpec((_NC, BN, D), lambda i: (0, i, 0)),
            pl.BlockSpec((D, D), lambda i: (0, 0)),
            pl.BlockSpec((1, D), lambda i: (0, 0)),
            pl.BlockSpec((1, D), lambda i: (0, 0)),
            pl.BlockSpec((1, D), lambda i: (0, 0)),
        ],
        out_specs=[
            pl.BlockSpec((BN, D), lambda i: (i, 0)),
            pl.BlockSpec((BN, D), lambda i: (i, 0)),
        ],
        out_shape=[
            jax.ShapeDtypeStruct((NP, D), jnp.float32),
            jax.ShapeDtypeStruct((NP, D), jnp.float32),
        ],
    )


def kernel(x, edge_index, num_rounds, W, b, gamma, beta):
    N, D = x.shape
    E = edge_index.shape[1]
    ALIGN = 1024
    NP = (N + ALIGN - 1) // ALIGN * ALIGN
    RP = NP // _NS

    src1 = edge_index[0].astype(jnp.int32)
    dst1 = edge_index[1].astype(jnp.int32)
    onesrows = jnp.ones((80, D), jnp.float32)
    zrows = jnp.zeros((RP, D), jnp.float32)
    xp = jnp.zeros((NP, D), jnp.float32).at[:N].set(x)
    b2 = b.reshape(1, D)
    g2 = gamma.reshape(1, D)
    be2 = beta.reshape(1, D)

    deg_kernel = _make_deg_kernel(NP, D, E)
    edge_kernel = _make_edge_kernel(NP, D, E)
    mm = _make_mm(NP, D)
    scale = _make_scale(NP, D)
    tc_round = _make_tc_round(NP, D)

    xw0 = mm(xp, W)
    deg = deg_kernel(dst1, onesrows, zrows)
    s0 = scale(deg, xw0)

    def round_body(_, carry):
        h, sarr = carry
        acc = edge_kernel(sarr, src1, dst1, zrows)
        h, sarr = tc_round(acc, deg, W, b2, g2, be2)
        return (h, sarr)

    h, _ = lax.fori_loop(0, num_rounds, round_body, (xp, s0))
    return h[:N]
